# baseline (speedup 1.0000x reference)
import functools

import jax
import jax.numpy as jnp
from jax import lax
from jax.experimental import pallas as pl
from jax.experimental.pallas import tpu as pltpu
from jax.experimental.pallas import tpu_sc as plsc

N = 10000
E = 160000
D = 256
DE = 16
H = 128
NC = 2
NS = 16
LANES = 16

K = 64
NCHUNKS = E // K
CH = NCHUNKS // NS
NEXTRA = NCHUNKS - CH * NS
_PIPELINE = True
R_A = 632
R_B = N - (NS - 1) * R_A


def _edge_proj_body(ea_ref, we_ref, be_ref, x_ref, eproj_ref, xsplit_ref):
    eproj_ref[...] = (
        jnp.dot(ea_ref[...], we_ref[...], preferred_element_type=jnp.float32)
        + be_ref[0]
    )
    xsplit_ref[...] = x_ref[...]


BE = 6400
NBE = E // BE
BXN = N // NBE


def _edge_proj(edge_attr, W_edge, b_edge, x):
    b3d = b_edge.reshape(NC, 1, H)
    return pl.pallas_call(
        _edge_proj_body,
        grid=(NC, NBE),
        in_specs=[
            pl.BlockSpec((BE, DE), lambda c, j: (j, 0)),
            pl.BlockSpec((DE, H), lambda c, j: (0, c)),
            pl.BlockSpec((1, 1, H), lambda c, j: (c, 0, 0)),
            pl.BlockSpec((BXN, H), lambda c, j: (j, c)),
        ],
        out_specs=[
            pl.BlockSpec((BE, H), lambda c, j: (c * NBE + j, 0)),
            pl.BlockSpec((BXN, H), lambda c, j: (c * NBE + j, 0)),
        ],
        out_shape=[
            jax.ShapeDtypeStruct((NC * E, H), jnp.float32),
            jax.ShapeDtypeStruct((NC * N, H), jnp.float32),
        ],
    )(edge_attr, W_edge, b3d, x)


def _sc_aggregate_body(x_hbm, eproj_hbm, src_hbm, dst_hbm, out_hbm,
                       accum, *rest):
    sidx = rest[0:3]
    didx = rest[3:6]
    bmsg = rest[6:9]
    bep = rest[9:12]
    gsem = rest[12:15]
    esem = rest[15:18]
    ssem = rest[18:21]
    isem = rest[21:24]
    dsem = rest[24:27]
    cid = lax.axis_index("c")
    sid = lax.axis_index("s")

    row0 = sid * R_A

    @pl.when(sid < NS - 1)
    def _():
        pltpu.sync_copy(x_hbm.at[pl.ds(cid * N + row0, R_A)],
                        accum.at[pl.ds(row0, R_A)])

    @pl.when(sid == NS - 1)
    def _():
        pltpu.sync_copy(x_hbm.at[pl.ds(cid * N + row0, R_B)],
                        accum.at[pl.ds(row0, R_B)])

    plsc.subcore_barrier()

    def load_sidx(c, b):
        pltpu.async_copy(src_hbm.at[pl.ds(cid * E + c * K, K)], sidx[b],
                         isem[b])

    def wait_sidx(b):
        pltpu.make_async_copy(src_hbm.at[pl.ds(0, K)], sidx[b],
                              isem[b]).wait()

    def load_didx(c, b):
        pltpu.async_copy(dst_hbm.at[pl.ds(c * K, K)], didx[b], dsem[b])

    def wait_didx(b):
        pltpu.make_async_copy(dst_hbm.at[pl.ds(0, K)], didx[b],
                              dsem[b]).wait()

    def load_main(c, b):
        pltpu.async_copy(x_hbm.at[sidx[b]], bmsg[b], gsem[b])
        pltpu.async_copy(eproj_hbm.at[pl.ds(cid * E + c * K, K)], bep[b],
                         esem[b])

    def wait_main(b):
        pltpu.make_async_copy(x_hbm.at[sidx[b]], bmsg[b], gsem[b]).wait()
        pltpu.make_async_copy(eproj_hbm.at[pl.ds(0, K)], bep[b],
                              esem[b]).wait()

    def issue_scatter(b):
        pltpu.async_copy(bmsg[b], accum.at[didx[b]], ssem[b], add=True)

    def wait_scatter(b):
        pltpu.make_async_copy(bmsg[b], accum.at[didx[b]], ssem[b]).wait()

    def compute(b):
        @plsc.parallel_loop(0, K, 1, unroll=4)
        def _(r):
            for jj in range(H // LANES):
                sl = pl.ds(jj * LANES, LANES)
                bmsg[b][r, sl] = jnp.maximum(bmsg[b][r, sl] + bep[b][r, sl],
                                             0.0)

    def slot(c, b, prologue_slot):
        b1 = (b + 1) % 3
        b2 = (b + 2) % 3
        if not prologue_slot:
            wait_scatter(b1)
        load_didx(c + 1, b1)
        load_sidx(c + 2, b2)
        wait_sidx(b1)
        load_main(c + 1, b1)
        wait_main(b)
        compute(b)
        wait_didx(b)
        issue_scatter(b)

    c0 = sid * CH
    if _PIPELINE:
        load_sidx(c0, 0)
        load_sidx(c0 + 1, 1)
        load_didx(c0, 0)
        wait_sidx(0)
        load_main(c0, 0)
        slot(c0, 0, True)
        slot(c0 + 1, 1, True)
        slot(c0 + 2, 2, False)

        def loop_body(t, carry):
            c = c0 + 3 + 3 * t
            slot(c, 0, False)
            slot(c + 1, 1, False)
            slot(c + 2, 2, False)
            return carry

        lax.fori_loop(0, (CH - 3) // 3, loop_body, 0)

        wait_scatter((CH - 2) % 3)
        wait_scatter((CH - 1) % 3)
        wait_main(CH % 3)
        wait_didx(CH % 3)
        wait_sidx((CH + 1) % 3)
    else:
        def sync_body(j, carry):
            c = c0 + j
            load_sidx(c, 0)
            load_didx(c, 0)
            wait_sidx(0)
            load_main(c, 0)
            wait_main(0)
            compute(0)
            wait_didx(0)
            issue_scatter(0)
            wait_scatter(0)
            return carry

        lax.fori_loop(0, CH, sync_body, 0)

    @pl.when(sid < NEXTRA)
    def _():
        c = NS * CH + sid
        load_sidx(c, 0)
        load_didx(c, 0)
        wait_sidx(0)
        load_main(c, 0)
        wait_main(0)
        compute(0)
        wait_didx(0)
        issue_scatter(0)
        wait_scatter(0)

    plsc.subcore_barrier()

    @pl.when(sid < NS - 1)
    def _():
        pltpu.sync_copy(accum.at[pl.ds(row0, R_A)],
                        out_hbm.at[pl.ds(cid * N + row0, R_A)])

    @pl.when(sid == NS - 1)
    def _():
        pltpu.sync_copy(accum.at[pl.ds(row0, R_B)],
                        out_hbm.at[pl.ds(cid * N + row0, R_B)])


_sc_aggregate = functools.partial(
    pl.kernel,
    out_type=jax.ShapeDtypeStruct((NC * N, H), jnp.float32),
    mesh=plsc.VectorSubcoreMesh(core_axis_name="c", subcore_axis_name="s",
                                num_cores=NC, num_subcores=NS),
    compiler_params=pltpu.CompilerParams(use_tc_tiling_on_sc=True),
    scratch_types=(
        [pltpu.VMEM_SHARED((N, H), jnp.float32)]
        + [pltpu.VMEM((K,), jnp.int32) for _ in range(6)]
        + [pltpu.VMEM((K, H), jnp.float32) for _ in range(6)]
        + [pltpu.SemaphoreType.DMA for _ in range(15)]
    ),
)(_sc_aggregate_body)


def _mlp_body(h0_ref, h1_ref, w1_ref, b1_ref, w2_ref, b2_ref, out_ref):
    w1 = w1_ref[...]
    t = (
        jnp.dot(h0_ref[...], w1[0:H, :], preferred_element_type=jnp.float32)
        + jnp.dot(h1_ref[...], w1[H:D, :], preferred_element_type=jnp.float32)
        + b1_ref[...]
    )
    t = jnp.maximum(t, 0.0)
    out_ref[...] = (
        jnp.dot(t, w2_ref[...], preferred_element_type=jnp.float32)
        + b2_ref[...]
    )


BN = 1000
NBN = N // BN


def _mlp(h_split, W1, b1, W2, b2):
    return pl.pallas_call(
        _mlp_body,
        grid=(NBN,),
        in_specs=[
            pl.BlockSpec((BN, H), lambda i: (i, 0)),
            pl.BlockSpec((BN, H), lambda i: (NBN + i, 0)),
            pl.BlockSpec((D, 2 * D), lambda i: (0, 0)),
            pl.BlockSpec((1, 2 * D), lambda i: (0, 0)),
            pl.BlockSpec((2 * D, D), lambda i: (0, 0)),
            pl.BlockSpec((1, D), lambda i: (0, 0)),
        ],
        out_specs=pl.BlockSpec((BN, D), lambda i: (i, 0)),
        out_shape=jax.ShapeDtypeStruct((N, D), jnp.float32),
    )(h_split, h_split, W1, b1.reshape(1, 2 * D), W2, b2.reshape(1, D))


def kernel(x, edge_index, edge_attr, W_edge, b_edge, W1, b1, W2, b2):
    src = edge_index[0]
    dst = edge_index[1]
    src2 = jnp.concatenate([src, src + N])
    eproj, x_split = _edge_proj(edge_attr, W_edge, b_edge, x)
    h_split = _sc_aggregate(x_split, eproj, src2, dst)
    return _mlp(h_split, W1, b1, W2, b2)


# speedup vs baseline: 1.2181x; 1.2181x over previous
import functools

import jax
import jax.numpy as jnp
from jax import lax
from jax.experimental import pallas as pl
from jax.experimental.pallas import tpu as pltpu
from jax.experimental.pallas import tpu_sc as plsc

N = 10000
E = 160000
D = 256
DE = 16
H = 128
NC = 2
NS = 16
LANES = 16

K = 64
NCHUNKS = E // K
CH = NCHUNKS // NS
NEXTRA = NCHUNKS - CH * NS
_PIPELINE = True
R_A = 632
R_B = N - (NS - 1) * R_A


def _edge_proj_body(ea_ref, we_ref, be_ref, x_ref, eproj_ref, xsplit_ref):
    eproj_ref[...] = (
        lax.dot_general(ea_ref[...], we_ref[...],
                        (((0,), (0,)), ((), ())),
                        preferred_element_type=jnp.float32)
        + be_ref[0]
    )
    xsplit_ref[...] = x_ref[...]


BE = 6400
NBE = E // BE
BXN = N // NBE


def _edge_proj(edge_attr_t, W_edge, b_edge, x):
    b3d = b_edge.reshape(NC, 1, H)
    return pl.pallas_call(
        _edge_proj_body,
        grid=(NC, NBE),
        in_specs=[
            pl.BlockSpec((DE, BE), lambda c, j: (0, j)),
            pl.BlockSpec((DE, H), lambda c, j: (0, c)),
            pl.BlockSpec((1, 1, H), lambda c, j: (c, 0, 0)),
            pl.BlockSpec((BXN, H), lambda c, j: (j, c)),
        ],
        out_specs=[
            pl.BlockSpec((BE, H), lambda c, j: (c * NBE + j, 0)),
            pl.BlockSpec((BXN, H), lambda c, j: (c * NBE + j, 0)),
        ],
        out_shape=[
            jax.ShapeDtypeStruct((NC * E, H), jnp.float32),
            jax.ShapeDtypeStruct((NC * N, H), jnp.float32),
        ],
    )(edge_attr_t, W_edge, b3d, x)


def _sc_aggregate_body(x_hbm, eproj_hbm, src_hbm, dst_hbm, out_hbm,
                       accum, *rest):
    sidx = rest[0:3]
    didx = rest[3:6]
    bmsg = rest[6:9]
    bep = rest[9:12]
    gsem = rest[12:15]
    esem = rest[15:18]
    ssem = rest[18:21]
    isem = rest[21:24]
    dsem = rest[24:27]
    cid = lax.axis_index("c")
    sid = lax.axis_index("s")

    row0 = sid * R_A

    @pl.when(sid < NS - 1)
    def _():
        pltpu.sync_copy(x_hbm.at[pl.ds(cid * N + row0, R_A)],
                        accum.at[pl.ds(row0, R_A)])

    @pl.when(sid == NS - 1)
    def _():
        pltpu.sync_copy(x_hbm.at[pl.ds(cid * N + row0, R_B)],
                        accum.at[pl.ds(row0, R_B)])

    plsc.subcore_barrier()

    def load_sidx(c, b):
        pltpu.async_copy(src_hbm.at[pl.ds(cid * E + c * K, K)], sidx[b],
                         isem[b])

    def wait_sidx(b):
        pltpu.make_async_copy(src_hbm.at[pl.ds(0, K)], sidx[b],
                              isem[b]).wait()

    def load_didx(c, b):
        pltpu.async_copy(dst_hbm.at[pl.ds(c * K, K)], didx[b], dsem[b])

    def wait_didx(b):
        pltpu.make_async_copy(dst_hbm.at[pl.ds(0, K)], didx[b],
                              dsem[b]).wait()

    def load_main(c, b):
        pltpu.async_copy(x_hbm.at[sidx[b]], bmsg[b], gsem[b])
        pltpu.async_copy(eproj_hbm.at[pl.ds(cid * E + c * K, K)], bep[b],
                         esem[b])

    def wait_main(b):
        pltpu.make_async_copy(x_hbm.at[sidx[b]], bmsg[b], gsem[b]).wait()
        pltpu.make_async_copy(eproj_hbm.at[pl.ds(0, K)], bep[b],
                              esem[b]).wait()

    def issue_scatter(b):
        pltpu.async_copy(bmsg[b], accum.at[didx[b]], ssem[b], add=True)

    def wait_scatter(b):
        pltpu.make_async_copy(bmsg[b], accum.at[didx[b]], ssem[b]).wait()

    def compute(b):
        @plsc.parallel_loop(0, K, 1, unroll=4)
        def _(r):
            for jj in range(H // LANES):
                sl = pl.ds(jj * LANES, LANES)
                bmsg[b][r, sl] = jnp.maximum(bmsg[b][r, sl] + bep[b][r, sl],
                                             0.0)

    def slot(c, b, prologue_slot):
        b1 = (b + 1) % 3
        b2 = (b + 2) % 3
        if not prologue_slot:
            wait_scatter(b1)
        load_didx(c + 1, b1)
        load_sidx(c + 2, b2)
        wait_sidx(b1)
        load_main(c + 1, b1)
        wait_main(b)
        compute(b)
        wait_didx(b)
        issue_scatter(b)

    c0 = sid * CH
    if _PIPELINE:
        load_sidx(c0, 0)
        load_sidx(c0 + 1, 1)
        load_didx(c0, 0)
        wait_sidx(0)
        load_main(c0, 0)
        slot(c0, 0, True)
        slot(c0 + 1, 1, True)
        slot(c0 + 2, 2, False)

        def loop_body(t, carry):
            c = c0 + 3 + 3 * t
            slot(c, 0, False)
            slot(c + 1, 1, False)
            slot(c + 2, 2, False)
            return carry

        lax.fori_loop(0, (CH - 3) // 3, loop_body, 0)

        wait_scatter((CH - 2) % 3)
        wait_scatter((CH - 1) % 3)
        wait_main(CH % 3)
        wait_didx(CH % 3)
        wait_sidx((CH + 1) % 3)
    else:
        def sync_body(j, carry):
            c = c0 + j
            load_sidx(c, 0)
            load_didx(c, 0)
            wait_sidx(0)
            load_main(c, 0)
            wait_main(0)
            compute(0)
            wait_didx(0)
            issue_scatter(0)
            wait_scatter(0)
            return carry

        lax.fori_loop(0, CH, sync_body, 0)

    @pl.when(sid < NEXTRA)
    def _():
        c = NS * CH + sid
        load_sidx(c, 0)
        load_didx(c, 0)
        wait_sidx(0)
        load_main(c, 0)
        wait_main(0)
        compute(0)
        wait_didx(0)
        issue_scatter(0)
        wait_scatter(0)

    plsc.subcore_barrier()

    @pl.when(sid < NS - 1)
    def _():
        pltpu.sync_copy(accum.at[pl.ds(row0, R_A)],
                        out_hbm.at[pl.ds(cid * N + row0, R_A)])

    @pl.when(sid == NS - 1)
    def _():
        pltpu.sync_copy(accum.at[pl.ds(row0, R_B)],
                        out_hbm.at[pl.ds(cid * N + row0, R_B)])


_sc_aggregate = functools.partial(
    pl.kernel,
    out_type=jax.ShapeDtypeStruct((NC * N, H), jnp.float32),
    mesh=plsc.VectorSubcoreMesh(core_axis_name="c", subcore_axis_name="s",
                                num_cores=NC, num_subcores=NS),
    compiler_params=pltpu.CompilerParams(use_tc_tiling_on_sc=True),
    scratch_types=(
        [pltpu.VMEM_SHARED((N, H), jnp.float32)]
        + [pltpu.VMEM((K,), jnp.int32) for _ in range(6)]
        + [pltpu.VMEM((K, H), jnp.float32) for _ in range(6)]
        + [pltpu.SemaphoreType.DMA for _ in range(15)]
    ),
)(_sc_aggregate_body)


def _mlp_body(h0_ref, h1_ref, w1_ref, b1_ref, w2_ref, b2_ref, out_ref):
    w1 = w1_ref[...]
    t = (
        jnp.dot(h0_ref[...], w1[0:H, :], preferred_element_type=jnp.float32)
        + jnp.dot(h1_ref[...], w1[H:D, :], preferred_element_type=jnp.float32)
        + b1_ref[...]
    )
    t = jnp.maximum(t, 0.0)
    out_ref[...] = (
        jnp.dot(t, w2_ref[...], preferred_element_type=jnp.float32)
        + b2_ref[...]
    )


BN = 1000
NBN = N // BN


def _mlp(h_split, W1, b1, W2, b2):
    return pl.pallas_call(
        _mlp_body,
        grid=(NBN,),
        in_specs=[
            pl.BlockSpec((BN, H), lambda i: (i, 0)),
            pl.BlockSpec((BN, H), lambda i: (NBN + i, 0)),
            pl.BlockSpec((D, 2 * D), lambda i: (0, 0)),
            pl.BlockSpec((1, 2 * D), lambda i: (0, 0)),
            pl.BlockSpec((2 * D, D), lambda i: (0, 0)),
            pl.BlockSpec((1, D), lambda i: (0, 0)),
        ],
        out_specs=pl.BlockSpec((BN, D), lambda i: (i, 0)),
        out_shape=jax.ShapeDtypeStruct((N, D), jnp.float32),
    )(h_split, h_split, W1, b1.reshape(1, 2 * D), W2, b2.reshape(1, D))


def kernel(x, edge_index, edge_attr, W_edge, b_edge, W1, b1, W2, b2):
    src = edge_index[0]
    dst = edge_index[1]
    src2 = jnp.concatenate([src, src + N])
    eproj, x_split = _edge_proj(edge_attr.T, W_edge, b_edge, x)
    h_split = _sc_aggregate(x_split, eproj, src2, dst)
    return _mlp(h_split, W1, b1, W2, b2)
